# baseline (speedup 1.0000x reference)
import functools

import jax
import jax.numpy as jnp
from jax import lax
from jax.experimental import pallas as pl
from jax.experimental.pallas import tpu as pltpu
from jax.experimental.pallas import tpu_sc as plsc

N_NODES = 10000
D = 128
NC = 2
NS = 16
NW = NC * NS
W_EDGES = 40
DEG_W = 16
NBUF = 5
CHUNK = 125


def _sc_aggregate(x, ei3):
    n_win_total = ei3.shape[1]
    assert n_win_total % NW == 0
    n_win = n_win_total // NW
    assert n_win % CHUNK == 0
    n_chunk = n_win // CHUNK
    rows_per_sub = N_NODES // NS
    mesh = plsc.VectorSubcoreMesh(core_axis_name="c", subcore_axis_name="s")

    @functools.partial(
        pl.kernel,
        out_type=(
            jax.ShapeDtypeStruct((NC, N_NODES, D), jnp.float32),
            jax.ShapeDtypeStruct((NC, N_NODES, DEG_W), jnp.float32),
        ),
        mesh=mesh,
        scratch_types=[
            pltpu.VMEM_SHARED((N_NODES, D), jnp.float32),
            pltpu.VMEM_SHARED((N_NODES, DEG_W), jnp.float32),
            pltpu.VMEM((CHUNK, W_EDGES), jnp.int32),
            pltpu.VMEM((CHUNK, W_EDGES), jnp.int32),
            pltpu.VMEM((NBUF, W_EDGES, D), jnp.float32),
            pltpu.VMEM((W_EDGES, DEG_W), jnp.float32),
            pltpu.VMEM((125, DEG_W), jnp.float32),
            pltpu.SemaphoreType.DMA((NBUF,)),
            pltpu.SemaphoreType.DMA((NBUF,)),
            pltpu.SemaphoreType.DMA((NBUF,)),
        ],
        compiler_params=pltpu.CompilerParams(use_tc_tiling_on_sc=False),
    )
    def agg(x_hbm, ei_hbm, acc_out, deg_out,
            acc_sh, deg_sh, srcb, dstb, rows, ones, z16,
            sem_g, sem_s, sem_d):
        cid = lax.axis_index("c")
        sid = lax.axis_index("s")
        wid = cid * NS + sid
        zf = jnp.zeros((16,), jnp.float32)
        of = jnp.ones((16,), jnp.float32)

        for b in range(NBUF):
            @pl.loop(0, W_EDGES)
            def _(i):
                @pl.loop(0, D // 16)
                def _(j):
                    rows[b, i, pl.ds(j * 16, 16)] = zf

        @pl.loop(0, 125)
        def _(i):
            z16[i, :] = zf

        @pl.loop(0, W_EDGES)
        def _(i):
            ones[i, :] = of

        r0 = sid * rows_per_sub

        @pl.loop(0, rows_per_sub // W_EDGES)
        def _(j):
            pltpu.sync_copy(rows.at[0],
                            acc_sh.at[pl.ds(r0 + j * W_EDGES, W_EDGES)])

        tail = rows_per_sub % W_EDGES
        if tail:
            pltpu.sync_copy(
                rows.at[0, pl.ds(0, tail)],
                acc_sh.at[pl.ds(r0 + rows_per_sub - tail, tail)])

        @pl.loop(0, rows_per_sub // 125)
        def _(j):
            pltpu.sync_copy(z16, deg_sh.at[pl.ds(r0 + j * 125, 125)])

        plsc.subcore_barrier()

        base = wid * n_win

        @pl.loop(0, n_chunk)
        def _(c):
            pltpu.sync_copy(ei_hbm.at[0, pl.ds(base + c * CHUNK, CHUNK)], srcb)
            pltpu.sync_copy(ei_hbm.at[1, pl.ds(base + c * CHUNK, CHUNK)], dstb)

            for b in range(NBUF):
                pltpu.async_copy(x_hbm.at[srcb.at[b]], rows.at[b],
                                 sem_g.at[b])

            @pl.loop(0, CHUNK, step=NBUF)
            def _(o):
                for b in range(NBUF):
                    w = o + b
                    pltpu.make_async_copy(
                        x_hbm.at[srcb.at[b]], rows.at[b], sem_g.at[b]).wait()
                    pltpu.async_copy(
                        rows.at[b], acc_sh.at[dstb.at[w]], sem_s.at[b],
                        add=True)

                    @pl.when((c > 0) | (o > 0))
                    def _():
                        pltpu.make_async_copy(
                            ones, deg_sh.at[dstb.at[w]], sem_d.at[b]).wait()

                    pltpu.async_copy(
                        ones, deg_sh.at[dstb.at[w]], sem_d.at[b], add=True)
                    pltpu.make_async_copy(
                        rows.at[b], acc_sh.at[dstb.at[w]], sem_s.at[b]).wait()

                    @pl.when(w + NBUF < CHUNK)
                    def _():
                        pltpu.async_copy(
                            x_hbm.at[srcb.at[w + NBUF]], rows.at[b],
                            sem_g.at[b])

        for b in range(NBUF):
            pltpu.make_async_copy(ones, deg_sh.at[dstb.at[b]],
                                  sem_d.at[b]).wait()

        plsc.subcore_barrier()
        pltpu.sync_copy(acc_sh.at[pl.ds(r0, rows_per_sub)],
                        acc_out.at[cid, pl.ds(r0, rows_per_sub)])
        pltpu.sync_copy(deg_sh.at[pl.ds(r0, rows_per_sub)],
                        deg_out.at[cid, pl.ds(r0, rows_per_sub)])

    return agg(x, ei3)


def _tc_self(x, w_self, b2d):
    B = 2000
    grid = (N_NODES // B,)

    def body(x_ref, ws_ref, b_ref, o_ref):
        o_ref[...] = jnp.dot(x_ref[...], ws_ref[...],
                             preferred_element_type=jnp.float32) + b_ref[...]

    return pl.pallas_call(
        body,
        grid=grid,
        in_specs=[
            pl.BlockSpec((B, D), lambda i: (i, 0)),
            pl.BlockSpec((D, D), lambda i: (0, 0)),
            pl.BlockSpec((1, D), lambda i: (0, 0)),
        ],
        out_specs=pl.BlockSpec((B, D), lambda i: (i, 0)),
        out_shape=jax.ShapeDtypeStruct((N_NODES, D), jnp.float32),
    )(x, w_self, b2d)


def _tc_neigh(pre, acc, db128, w_neigh):
    B = 2000
    grid = (N_NODES // B,)

    def body(pre_ref, acc_ref, db_ref, wn_ref, o_ref):
        h = (acc_ref[0] + acc_ref[1]) / db_ref[...]
        o_ref[...] = pre_ref[...] + jnp.dot(
            h, wn_ref[...], preferred_element_type=jnp.float32)

    return pl.pallas_call(
        body,
        grid=grid,
        in_specs=[
            pl.BlockSpec((B, D), lambda i: (i, 0)),
            pl.BlockSpec((NC, B, D), lambda i: (0, i, 0)),
            pl.BlockSpec((B, D), lambda i: (i, 0)),
            pl.BlockSpec((D, D), lambda i: (0, 0)),
        ],
        out_specs=pl.BlockSpec((B, D), lambda i: (i, 0)),
        out_shape=jax.ShapeDtypeStruct((N_NODES, D), jnp.float32),
    )(pre, acc, db128, w_neigh)


def kernel(x, edge_index, W_self, W_neigh, b):
    ei3 = edge_index.astype(jnp.int32).reshape(2, -1, W_EDGES)
    acc, deg = _sc_aggregate(x, ei3)
    db = jnp.maximum(deg[0, :, 0] + deg[1, :, 0], 1.0)
    db128 = jnp.broadcast_to(db[:, None], (N_NODES, D))
    pre = _tc_self(x, W_self, b.reshape(1, D))
    return _tc_neigh(pre, acc, db128, W_neigh)


# speedup vs baseline: 1.0642x; 1.0642x over previous
import functools

import jax
import jax.numpy as jnp
from jax import lax
from jax.experimental import pallas as pl
from jax.experimental.pallas import tpu as pltpu
from jax.experimental.pallas import tpu_sc as plsc

N_NODES = 10000
D = 128
NC = 2
NS = 16
NW = NC * NS
W_EDGES = 40
DEG_W = 16
NBUF = 5
CHUNK = 125


def _sc_aggregate(x, ei3):
    n_win_total = ei3.shape[1]
    assert n_win_total % NW == 0
    n_win = n_win_total // NW
    assert n_win % CHUNK == 0
    n_chunk = n_win // CHUNK
    rows_per_sub = N_NODES // NS
    mesh = plsc.VectorSubcoreMesh(core_axis_name="c", subcore_axis_name="s")

    @functools.partial(
        pl.kernel,
        out_type=(
            jax.ShapeDtypeStruct((NC, N_NODES, D), jnp.float32),
            jax.ShapeDtypeStruct((NC, N_NODES, DEG_W), jnp.float32),
        ),
        mesh=mesh,
        scratch_types=[
            pltpu.VMEM_SHARED((N_NODES, D), jnp.float32),
            pltpu.VMEM_SHARED((N_NODES, DEG_W), jnp.float32),
            pltpu.VMEM((CHUNK, W_EDGES), jnp.int32),
            pltpu.VMEM((CHUNK, W_EDGES), jnp.int32),
            pltpu.VMEM((NBUF, W_EDGES, D), jnp.float32),
            pltpu.VMEM((W_EDGES, DEG_W), jnp.float32),
            pltpu.VMEM((125, DEG_W), jnp.float32),
            pltpu.SemaphoreType.DMA((NBUF,)),
            pltpu.SemaphoreType.DMA((NBUF,)),
            pltpu.SemaphoreType.DMA((NBUF,)),
        ],
        compiler_params=pltpu.CompilerParams(use_tc_tiling_on_sc=False),
    )
    def agg(x_hbm, ei_hbm, acc_out, deg_out,
            acc_sh, deg_sh, srcb, dstb, rows, ones, z16,
            sem_g, sem_s, sem_d):
        cid = lax.axis_index("c")
        sid = lax.axis_index("s")
        wid = cid * NS + sid
        zf = jnp.zeros((16,), jnp.float32)
        of = jnp.ones((16,), jnp.float32)

        for b in range(NBUF):
            @pl.loop(0, W_EDGES)
            def _(i):
                @pl.loop(0, D // 16)
                def _(j):
                    rows[b, i, pl.ds(j * 16, 16)] = zf

        @pl.loop(0, 125)
        def _(i):
            z16[i, :] = zf

        @pl.loop(0, W_EDGES)
        def _(i):
            ones[i, :] = of

        r0 = sid * rows_per_sub

        @pl.loop(0, rows_per_sub // W_EDGES)
        def _(j):
            pltpu.sync_copy(rows.at[0],
                            acc_sh.at[pl.ds(r0 + j * W_EDGES, W_EDGES)])

        tail = rows_per_sub % W_EDGES
        if tail:
            pltpu.sync_copy(
                rows.at[0, pl.ds(0, tail)],
                acc_sh.at[pl.ds(r0 + rows_per_sub - tail, tail)])

        @pl.loop(0, rows_per_sub // 125)
        def _(j):
            pltpu.sync_copy(z16, deg_sh.at[pl.ds(r0 + j * 125, 125)])

        plsc.subcore_barrier()

        base = wid * n_win

        @pl.loop(0, n_chunk)
        def _(c):
            pltpu.sync_copy(ei_hbm.at[0, pl.ds(base + c * CHUNK, CHUNK)], srcb)
            pltpu.sync_copy(ei_hbm.at[1, pl.ds(base + c * CHUNK, CHUNK)], dstb)

            for b in range(NBUF):
                pltpu.async_copy(x_hbm.at[srcb.at[b]], rows.at[b],
                                 sem_g.at[b])

            @pl.loop(0, CHUNK, step=NBUF)
            def _(o):
                for b in range(NBUF):
                    w = o + b
                    pltpu.make_async_copy(
                        x_hbm.at[srcb.at[b]], rows.at[b], sem_g.at[b]).wait()
                    pltpu.async_copy(
                        rows.at[b], acc_sh.at[dstb.at[w]], sem_s.at[b],
                        add=True)

                    @pl.when((c > 0) | (o > 0))
                    def _():
                        pltpu.make_async_copy(
                            ones, deg_sh.at[dstb.at[w]], sem_d.at[b]).wait()

                    pltpu.async_copy(
                        ones, deg_sh.at[dstb.at[w]], sem_d.at[b], add=True)
                    pltpu.make_async_copy(
                        rows.at[b], acc_sh.at[dstb.at[w]], sem_s.at[b]).wait()

                    @pl.when(w + NBUF < CHUNK)
                    def _():
                        pltpu.async_copy(
                            x_hbm.at[srcb.at[w + NBUF]], rows.at[b],
                            sem_g.at[b])

        for b in range(NBUF):
            pltpu.make_async_copy(ones, deg_sh.at[dstb.at[b]],
                                  sem_d.at[b]).wait()

        plsc.subcore_barrier()
        pltpu.sync_copy(acc_sh.at[pl.ds(r0, rows_per_sub)],
                        acc_out.at[cid, pl.ds(r0, rows_per_sub)])
        pltpu.sync_copy(deg_sh.at[pl.ds(r0, rows_per_sub)],
                        deg_out.at[cid, pl.ds(r0, rows_per_sub)])

    return agg(x, ei3)


def _tc_self(x, w_self, b2d):
    B = 5000
    grid = (N_NODES // B,)

    def body(x_ref, ws_ref, b_ref, o_ref):
        o_ref[...] = jnp.dot(x_ref[...], ws_ref[...],
                             preferred_element_type=jnp.float32) + b_ref[...]

    return pl.pallas_call(
        body,
        grid=grid,
        in_specs=[
            pl.BlockSpec((B, D), lambda i: (i, 0)),
            pl.BlockSpec((D, D), lambda i: (0, 0)),
            pl.BlockSpec((1, D), lambda i: (0, 0)),
        ],
        out_specs=pl.BlockSpec((B, D), lambda i: (i, 0)),
        out_shape=jax.ShapeDtypeStruct((N_NODES, D), jnp.float32),
    )(x, w_self, b2d)


def _tc_neigh(pre, acc, deg, w_neigh):
    B = 5000
    grid = (N_NODES // B,)

    def body(pre_ref, acc_ref, deg_ref, wn_ref, o_ref):
        h = acc_ref[0] + acc_ref[1]
        dg = deg_ref[0, :, 0:1] + deg_ref[1, :, 0:1]
        h = h / jnp.maximum(dg, 1.0)
        o_ref[...] = pre_ref[...] + jnp.dot(
            h, wn_ref[...], preferred_element_type=jnp.float32)

    return pl.pallas_call(
        body,
        grid=grid,
        in_specs=[
            pl.BlockSpec((B, D), lambda i: (i, 0)),
            pl.BlockSpec((NC, B, D), lambda i: (0, i, 0)),
            pl.BlockSpec((NC, B, DEG_W), lambda i: (0, i, 0)),
            pl.BlockSpec((D, D), lambda i: (0, 0)),
        ],
        out_specs=pl.BlockSpec((B, D), lambda i: (i, 0)),
        out_shape=jax.ShapeDtypeStruct((N_NODES, D), jnp.float32),
    )(pre, acc, deg, w_neigh)


def kernel(x, edge_index, W_self, W_neigh, b):
    ei3 = edge_index.astype(jnp.int32).reshape(2, -1, W_EDGES)
    acc, deg = _sc_aggregate(x, ei3)
    pre = _tc_self(x, W_self, b.reshape(1, D))
    return _tc_neigh(pre, acc, deg, W_neigh)


# speedup vs baseline: 1.0679x; 1.0034x over previous
import functools

import jax
import jax.numpy as jnp
from jax import lax
from jax.experimental import pallas as pl
from jax.experimental.pallas import tpu as pltpu
from jax.experimental.pallas import tpu_sc as plsc

N_NODES = 10000
D = 128
NC = 2
NS = 16
NW = NC * NS
W_EDGES = 40
DEG_W = 16
NBUF = 5
CHUNK = 125


def _sc_aggregate(x, ei3):
    n_win_total = ei3.shape[1]
    assert n_win_total % NW == 0
    n_win = n_win_total // NW
    assert n_win % CHUNK == 0
    n_chunk = n_win // CHUNK
    rows_per_sub = N_NODES // NS
    mesh = plsc.VectorSubcoreMesh(core_axis_name="c", subcore_axis_name="s")

    @functools.partial(
        pl.kernel,
        out_type=(
            jax.ShapeDtypeStruct((NC, N_NODES, D), jnp.float32),
            jax.ShapeDtypeStruct((NC, N_NODES, DEG_W), jnp.float32),
        ),
        mesh=mesh,
        scratch_types=[
            pltpu.VMEM_SHARED((N_NODES, D), jnp.float32),
            pltpu.VMEM_SHARED((N_NODES, DEG_W), jnp.float32),
            pltpu.VMEM((CHUNK, W_EDGES), jnp.int32),
            pltpu.VMEM((CHUNK, W_EDGES), jnp.int32),
            pltpu.VMEM((NBUF, W_EDGES, D), jnp.float32),
            pltpu.VMEM((W_EDGES, DEG_W), jnp.float32),
            pltpu.VMEM((125, DEG_W), jnp.float32),
            pltpu.SemaphoreType.DMA((NBUF,)),
            pltpu.SemaphoreType.DMA((NBUF,)),
            pltpu.SemaphoreType.DMA((NBUF,)),
        ],
        compiler_params=pltpu.CompilerParams(use_tc_tiling_on_sc=False),
    )
    def agg(x_hbm, ei_hbm, acc_out, deg_out,
            acc_sh, deg_sh, srcb, dstb, rows, ones, z16,
            sem_g, sem_s, sem_d):
        cid = lax.axis_index("c")
        sid = lax.axis_index("s")
        wid = cid * NS + sid
        zf = jnp.zeros((16,), jnp.float32)
        of = jnp.ones((16,), jnp.float32)

        for b in range(NBUF):
            @pl.loop(0, W_EDGES)
            def _(i):
                @pl.loop(0, D // 16)
                def _(j):
                    rows[b, i, pl.ds(j * 16, 16)] = zf

        @pl.loop(0, 125)
        def _(i):
            z16[i, :] = zf

        @pl.loop(0, W_EDGES)
        def _(i):
            ones[i, :] = of

        r0 = sid * rows_per_sub

        @pl.loop(0, rows_per_sub // W_EDGES)
        def _(j):
            pltpu.sync_copy(rows.at[0],
                            acc_sh.at[pl.ds(r0 + j * W_EDGES, W_EDGES)])

        tail = rows_per_sub % W_EDGES
        if tail:
            pltpu.sync_copy(
                rows.at[0, pl.ds(0, tail)],
                acc_sh.at[pl.ds(r0 + rows_per_sub - tail, tail)])

        @pl.loop(0, rows_per_sub // 125)
        def _(j):
            pltpu.sync_copy(z16, deg_sh.at[pl.ds(r0 + j * 125, 125)])

        plsc.subcore_barrier()

        base = wid * n_win

        @pl.loop(0, n_chunk)
        def _(c):
            pltpu.sync_copy(ei_hbm.at[0, pl.ds(base + c * CHUNK, CHUNK)], srcb)
            pltpu.sync_copy(ei_hbm.at[1, pl.ds(base + c * CHUNK, CHUNK)], dstb)

            for b in range(NBUF):
                pltpu.async_copy(x_hbm.at[srcb.at[b]], rows.at[b],
                                 sem_g.at[b])

            @pl.loop(0, CHUNK, step=NBUF)
            def _(o):
                for b in range(NBUF):
                    w = o + b
                    pltpu.make_async_copy(
                        x_hbm.at[srcb.at[b]], rows.at[b], sem_g.at[b]).wait()
                    pltpu.async_copy(
                        rows.at[b], acc_sh.at[dstb.at[w]], sem_s.at[b],
                        add=True)

                    @pl.when((c > 0) | (o > 0))
                    def _():
                        pltpu.make_async_copy(
                            ones, deg_sh.at[dstb.at[w]], sem_d.at[b]).wait()

                    pltpu.async_copy(
                        ones, deg_sh.at[dstb.at[w]], sem_d.at[b], add=True)
                    pltpu.make_async_copy(
                        rows.at[b], acc_sh.at[dstb.at[w]], sem_s.at[b]).wait()

                    @pl.when(w + NBUF < CHUNK)
                    def _():
                        pltpu.async_copy(
                            x_hbm.at[srcb.at[w + NBUF]], rows.at[b],
                            sem_g.at[b])

        for b in range(NBUF):
            pltpu.make_async_copy(ones, deg_sh.at[dstb.at[b]],
                                  sem_d.at[b]).wait()

        plsc.subcore_barrier()
        pltpu.sync_copy(acc_sh.at[pl.ds(r0, rows_per_sub)],
                        acc_out.at[cid, pl.ds(r0, rows_per_sub)])
        pltpu.sync_copy(deg_sh.at[pl.ds(r0, rows_per_sub)],
                        deg_out.at[cid, pl.ds(r0, rows_per_sub)])

    return agg(x, ei3)


def _tc_self(x, w_self, b2d):
    B = 2000
    grid = (N_NODES // B,)

    def body(x_ref, ws_ref, b_ref, o_ref):
        o_ref[...] = jnp.dot(x_ref[...], ws_ref[...],
                             preferred_element_type=jnp.float32) + b_ref[...]

    return pl.pallas_call(
        body,
        grid=grid,
        in_specs=[
            pl.BlockSpec((B, D), lambda i: (i, 0)),
            pl.BlockSpec((D, D), lambda i: (0, 0)),
            pl.BlockSpec((1, D), lambda i: (0, 0)),
        ],
        out_specs=pl.BlockSpec((B, D), lambda i: (i, 0)),
        out_shape=jax.ShapeDtypeStruct((N_NODES, D), jnp.float32),
    )(x, w_self, b2d)


def _tc_neigh(pre, acc, deg, w_neigh):
    B = 2000
    grid = (N_NODES // B,)

    def body(pre_ref, acc_ref, deg_ref, wn_ref, o_ref):
        h = acc_ref[0] + acc_ref[1]
        dg = deg_ref[0, :, 0:1] + deg_ref[1, :, 0:1]
        h = h / jnp.maximum(dg, 1.0)
        o_ref[...] = pre_ref[...] + jnp.dot(
            h, wn_ref[...], preferred_element_type=jnp.float32)

    return pl.pallas_call(
        body,
        grid=grid,
        in_specs=[
            pl.BlockSpec((B, D), lambda i: (i, 0)),
            pl.BlockSpec((NC, B, D), lambda i: (0, i, 0)),
            pl.BlockSpec((NC, B, DEG_W), lambda i: (0, i, 0)),
            pl.BlockSpec((D, D), lambda i: (0, 0)),
        ],
        out_specs=pl.BlockSpec((B, D), lambda i: (i, 0)),
        out_shape=jax.ShapeDtypeStruct((N_NODES, D), jnp.float32),
    )(pre, acc, deg, w_neigh)


def kernel(x, edge_index, W_self, W_neigh, b):
    ei3 = edge_index.astype(jnp.int32).reshape(2, -1, W_EDGES)
    acc, deg = _sc_aggregate(x, ei3)
    pre = _tc_self(x, W_self, b.reshape(1, D))
    return _tc_neigh(pre, acc, deg, W_neigh)
